# Initial kernel scaffold; baseline (speedup 1.0000x reference)
import jax
import jax.numpy as jnp
from jax.experimental import pallas as pl


def kernel(trainable, batch_vec):
    raise NotImplementedError("write your pallas kernel here")


# speedup vs baseline: 20.0450x; 20.0450x over previous
import jax
import jax.numpy as jnp
from jax.experimental import pallas as pl

_ROWS_PER_BLOCK = 2000


def _copy_block(x_ref, o_ref):
    o_ref[...] = x_ref[...]


def kernel(trainable, batch_vec):
    n, d = trainable.shape
    grid = pl.cdiv(n, _ROWS_PER_BLOCK)
    return pl.pallas_call(
        _copy_block,
        grid=(grid,),
        in_specs=[pl.BlockSpec((_ROWS_PER_BLOCK, d), lambda i: (i, 0))],
        out_specs=pl.BlockSpec((_ROWS_PER_BLOCK, d), lambda i: (i, 0)),
        out_shape=jax.ShapeDtypeStruct((n, d), trainable.dtype),
    )(trainable)


# speedup vs baseline: 30.9990x; 1.5465x over previous
import jax
import jax.numpy as jnp
from jax.experimental import pallas as pl

_ROWS_PER_BLOCK = 10000


def _copy_block(x_ref, o_ref):
    o_ref[...] = x_ref[...]


def kernel(trainable, batch_vec):
    n, d = trainable.shape
    grid = pl.cdiv(n, _ROWS_PER_BLOCK)
    return pl.pallas_call(
        _copy_block,
        grid=(grid,),
        in_specs=[pl.BlockSpec((_ROWS_PER_BLOCK, d), lambda i: (i, 0))],
        out_specs=pl.BlockSpec((_ROWS_PER_BLOCK, d), lambda i: (i, 0)),
        out_shape=jax.ShapeDtypeStruct((n, d), trainable.dtype),
    )(trainable)


# speedup vs baseline: 32.1045x; 1.0357x over previous
import jax
import jax.numpy as jnp
from jax.experimental import pallas as pl

_ROWS_PER_BLOCK = 20000


def _copy_block(x_ref, o_ref):
    o_ref[...] = x_ref[...]


def kernel(trainable, batch_vec):
    n, d = trainable.shape
    grid = pl.cdiv(n, _ROWS_PER_BLOCK)
    return pl.pallas_call(
        _copy_block,
        grid=(grid,),
        in_specs=[pl.BlockSpec((_ROWS_PER_BLOCK, d), lambda i: (i, 0))],
        out_specs=pl.BlockSpec((_ROWS_PER_BLOCK, d), lambda i: (i, 0)),
        out_shape=jax.ShapeDtypeStruct((n, d), trainable.dtype),
    )(trainable)


# speedup vs baseline: 32.2084x; 1.0032x over previous
import jax
import jax.numpy as jnp
from jax.experimental import pallas as pl

_ROWS_PER_BLOCK = 25000


def _copy_block(x_ref, o_ref):
    o_ref[...] = x_ref[...]


def kernel(trainable, batch_vec):
    n, d = trainable.shape
    grid = pl.cdiv(n, _ROWS_PER_BLOCK)
    return pl.pallas_call(
        _copy_block,
        grid=(grid,),
        in_specs=[pl.BlockSpec((_ROWS_PER_BLOCK, d), lambda i: (i, 0))],
        out_specs=pl.BlockSpec((_ROWS_PER_BLOCK, d), lambda i: (i, 0)),
        out_shape=jax.ShapeDtypeStruct((n, d), trainable.dtype),
    )(trainable)


# speedup vs baseline: 32.3126x; 1.0032x over previous
import jax
import jax.numpy as jnp
from jax.experimental import pallas as pl
from jax.experimental.pallas import tpu as pltpu

_ROWS_PER_BLOCK = 25000


def _copy_block(x_ref, o_ref):
    o_ref[...] = x_ref[...]


def kernel(trainable, batch_vec):
    n, d = trainable.shape
    grid = pl.cdiv(n, _ROWS_PER_BLOCK)
    return pl.pallas_call(
        _copy_block,
        grid=(grid,),
        in_specs=[pl.BlockSpec((_ROWS_PER_BLOCK, d), lambda i: (i, 0))],
        out_specs=pl.BlockSpec((_ROWS_PER_BLOCK, d), lambda i: (i, 0)),
        out_shape=jax.ShapeDtypeStruct((n, d), trainable.dtype),
        compiler_params=pltpu.CompilerParams(
            dimension_semantics=("parallel",)
        ),
    )(trainable)
